# initial kernel scaffold (unmeasured)
import jax
import jax.numpy as jnp
from jax import lax
from jax.experimental import pallas as pl
from jax.experimental.pallas import tpu as pltpu

N_DEV = 32
B_LOC = 2
SQ = 128
SKV = 128
HG = 4
DH = 64
D_MODEL = 512
D_CHUNK = HG * DH


def kernel(x, Wq, K_ext, V_ext, Wo):
    pos = lax.axis_index("i")

    def regroup(a):
        a = lax.dynamic_slice_in_dim(a, B_LOC * pos, B_LOC, axis=0)
        a = a.transpose(0, 2, 1, 3)
        a = a.reshape(B_LOC, N_DEV, HG, SKV, DH)
        a = a.transpose(0, 1, 3, 2, 4)
        return a.reshape(B_LOC, N_DEV, SKV, HG * DH)

    Kg = regroup(K_ext)
    Vg = regroup(V_ext)
    x2d = x.reshape(B_LOC * SQ, D_MODEL)
    packed = jnp.stack([Wq, Wo.T])

    def body(x_ref, w_ref, k_ref, v_ref, out_ref,
             comm_ref, ctx_ref, send_sems, recv_sems):
        my = lax.axis_index("i")
        left = lax.rem(my - 1 + N_DEV, N_DEV)
        right = lax.rem(my + 1, N_DEV)

        barrier = pltpu.get_barrier_semaphore()
        pl.semaphore_signal(barrier, inc=1, device_id=(left,),
                            device_id_type=pl.DeviceIdType.MESH)
        pl.semaphore_signal(barrier, inc=1, device_id=(right,),
                            device_id_type=pl.DeviceIdType.MESH)
        pl.semaphore_wait(barrier, 2)

        comm_ref[my] = w_ref[...]

        rowb = lax.broadcasted_iota(jnp.int32, (SQ, SKV), 0) // 64
        colb = lax.broadcasted_iota(jnp.int32, (SQ, SKV), 1) // 64
        mask = (rowb == colb) | (colb == 0) | (lax.rem(rowb + colb, 3) == 0)

        out_ref[...] = jnp.zeros_like(out_ref)

        def compute_chunk(s):
            wq = comm_ref[s, 0]
            wot = comm_ref[s, 1]
            q = lax.dot_general(x_ref[...], wq, (((1,), (0,)), ((), ())),
                                preferred_element_type=jnp.float32)
            for b in range(B_LOC):
                kblk = k_ref[b, s]
                vblk = v_ref[b, s]
                for h in range(HG):
                    qh = q[b * SQ:(b + 1) * SQ, h * DH:(h + 1) * DH]
                    kh = kblk[:, h * DH:(h + 1) * DH]
                    sc = lax.dot_general(
                        qh, kh, (((1,), (1,)), ((), ())),
                        preferred_element_type=jnp.float32) * 0.125
                    sc = jnp.where(mask, sc, -1e9)
                    m = jnp.max(sc, axis=1, keepdims=True)
                    w = jnp.exp(sc - m)
                    w = w / jnp.sum(w, axis=1, keepdims=True)
                    vh = vblk[:, h * DH:(h + 1) * DH]
                    c = lax.dot_general(w, vh, (((1,), (0,)), ((), ())),
                                        preferred_element_type=jnp.float32)
                    ctx_ref[b * SQ:(b + 1) * SQ, h * DH:(h + 1) * DH] = c
            out_ref[...] += lax.dot_general(
                ctx_ref[...], wot, (((1,), (1,)), ((), ())),
                preferred_element_type=jnp.float32)

        def hop(h, carry):
            s = lax.rem(my - h + N_DEV, N_DEV)
            rdma = pltpu.make_async_remote_copy(
                src_ref=comm_ref.at[s],
                dst_ref=comm_ref.at[s],
                send_sem=send_sems.at[h],
                recv_sem=recv_sems.at[h],
                device_id=(right,),
                device_id_type=pl.DeviceIdType.MESH,
            )
            rdma.start()
            compute_chunk(s)
            rdma.wait()
            return carry

        lax.fori_loop(0, N_DEV - 1, hop, 0)
        compute_chunk(lax.rem(my + 1, N_DEV))

    out2d = pl.pallas_call(
        body,
        out_shape=jax.ShapeDtypeStruct((B_LOC * SQ, D_MODEL), jnp.float32),
        in_specs=[pl.BlockSpec(memory_space=pltpu.VMEM)] * 4,
        out_specs=pl.BlockSpec(memory_space=pltpu.VMEM),
        scratch_shapes=[
            pltpu.VMEM((N_DEV, 2, D_MODEL, D_CHUNK), jnp.float32),
            pltpu.VMEM((B_LOC * SQ, D_CHUNK), jnp.float32),
            pltpu.SemaphoreType.DMA((N_DEV - 1,)),
            pltpu.SemaphoreType.DMA((N_DEV - 1,)),
        ],
        compiler_params=pltpu.CompilerParams(collective_id=0),
    )(x2d, packed, Kg, Vg)
    return out2d.reshape(B_LOC, SQ, D_MODEL)


# baseline (device time: 453625 ns/iter reference)
import jax
import jax.numpy as jnp
from jax import lax
from jax.experimental import pallas as pl
from jax.experimental.pallas import tpu as pltpu

N_DEV = 32
B_LOC = 2
SQ = 128
SKV = 128
HG = 4
DH = 64
D_MODEL = 512
D_CHUNK = HG * DH


def kernel(x, Wq, K_ext, V_ext, Wo):
    pos = lax.axis_index("i")

    def regroup(a):
        a = lax.dynamic_slice_in_dim(a, B_LOC * pos, B_LOC, axis=0)
        a = a.transpose(0, 2, 1, 3)
        a = a.reshape(B_LOC, N_DEV, HG, SKV, DH)
        a = a.transpose(0, 1, 3, 2, 4)
        return a.reshape(B_LOC, N_DEV, SKV, HG * DH)

    Kg = regroup(K_ext)
    Vg = regroup(V_ext)
    x2d = x.reshape(B_LOC * SQ, D_MODEL)
    packed = jnp.stack([Wq, Wo.T])

    def body(x_ref, w_ref, k_ref, v_ref, out_ref,
             comm_ref, ctx_ref, send_sems, recv_sems):
        my = lax.axis_index("i")
        left = lax.rem(my - 1 + N_DEV, N_DEV)
        right = lax.rem(my + 1, N_DEV)

        barrier = pltpu.get_barrier_semaphore()
        pl.semaphore_signal(barrier, inc=1, device_id=(left,),
                            device_id_type=pl.DeviceIdType.MESH)
        pl.semaphore_signal(barrier, inc=1, device_id=(right,),
                            device_id_type=pl.DeviceIdType.MESH)
        pl.semaphore_wait(barrier, 2)

        comm_ref[my] = w_ref[...]

        rowb = lax.broadcasted_iota(jnp.int32, (SQ, SKV), 0) // 64
        colb = lax.broadcasted_iota(jnp.int32, (SQ, SKV), 1) // 64
        mask = (rowb == colb) | (colb == 0) | (lax.rem(rowb + colb, 3) == 0)

        out_ref[...] = jnp.zeros_like(out_ref)

        def compute_chunk(s):
            wq = comm_ref[s, 0]
            wot = comm_ref[s, 1]
            q = lax.dot_general(x_ref[...], wq, (((1,), (0,)), ((), ())),
                                preferred_element_type=jnp.float32)
            for b in range(B_LOC):
                kblk = k_ref[b, s]
                vblk = v_ref[b, s]
                for h in range(HG):
                    qh = q[b * SQ:(b + 1) * SQ, h * DH:(h + 1) * DH]
                    kh = kblk[:, h * DH:(h + 1) * DH]
                    sc = lax.dot_general(
                        qh, kh, (((1,), (1,)), ((), ())),
                        preferred_element_type=jnp.float32) * 0.125
                    sc = jnp.where(mask, sc, -1e9)
                    m = jnp.max(sc, axis=1, keepdims=True)
                    w = jnp.exp(sc - m)
                    w = w / jnp.sum(w, axis=1, keepdims=True)
                    vh = vblk[:, h * DH:(h + 1) * DH]
                    c = lax.dot_general(w, vh, (((1,), (0,)), ((), ())),
                                        preferred_element_type=jnp.float32)
                    ctx_ref[b * SQ:(b + 1) * SQ, h * DH:(h + 1) * DH] = c
            out_ref[...] += lax.dot_general(
                ctx_ref[...], wot, (((1,), (1,)), ((), ())),
                preferred_element_type=jnp.float32)

        def hop(h, carry):
            s = lax.rem(my - h + N_DEV, N_DEV)
            rdma = pltpu.make_async_remote_copy(
                src_ref=comm_ref.at[s],
                dst_ref=comm_ref.at[s],
                send_sem=send_sems.at[h],
                recv_sem=recv_sems.at[h],
                device_id=(right,),
                device_id_type=pl.DeviceIdType.MESH,
            )
            rdma.start()
            compute_chunk(s)
            rdma.wait()
            return carry

        lax.fori_loop(0, N_DEV - 1, hop, 0)
        compute_chunk(lax.rem(my + 1, N_DEV))

    out2d = pl.pallas_call(
        body,
        out_shape=jax.ShapeDtypeStruct((B_LOC * SQ, D_MODEL), jnp.float32),
        in_specs=[pl.BlockSpec(memory_space=pltpu.VMEM)] * 4,
        out_specs=pl.BlockSpec(memory_space=pltpu.VMEM),
        scratch_shapes=[
            pltpu.VMEM((N_DEV, 2, D_MODEL, D_CHUNK), jnp.float32),
            pltpu.VMEM((B_LOC * SQ, D_CHUNK), jnp.float32),
            pltpu.SemaphoreType.DMA((N_DEV - 1,)),
            pltpu.SemaphoreType.DMA((N_DEV - 1,)),
        ],
        compiler_params=pltpu.CompilerParams(
            collective_id=0, vmem_limit_bytes=96 * 1024 * 1024),
    )(x2d, packed, Kg, Vg)
    return out2d.reshape(B_LOC, SQ, D_MODEL)
